# initial kernel scaffold (unmeasured)
import jax
import jax.numpy as jnp
from jax import lax
from jax.experimental import pallas as pl
from jax.experimental.pallas import tpu as pltpu

N_DEV = 8
N_TOK = 4096
D_MODEL = 1024
D_FF = 2048
E_LOCAL = 4
CHUNK = N_TOK // N_DEV


def kernel(x, router_W, route_idx, expert_W):
    x_bf = x.astype(jnp.bfloat16)
    w_bf = expert_W.astype(jnp.bfloat16)

    def body(
        x_ref,
        route_ref,
        w_ref,
        out_ref,
        acc_ref,
        rs_comm,
        ag_comm,
        rs_send_sems,
        rs_recv_sems,
        ag_send_sems,
        ag_recv_sems,
    ):
        d = lax.axis_index("i")
        right = lax.rem(d + 1, N_DEV)
        left = lax.rem(d + N_DEV - 1, N_DEV)

        barrier_sem = pltpu.get_barrier_semaphore()
        for nbr in (left, right):
            pl.semaphore_signal(
                barrier_sem,
                inc=1,
                device_id=(nbr,),
                device_id_type=pl.DeviceIdType.MESH,
            )
        pl.semaphore_wait(barrier_sem, 2)

        for c in range(N_DEV):
            rows = pl.ds(c * CHUNK, CHUNK)
            xc = x_ref[rows, :]
            rc = route_ref[rows, :]
            total = None
            for j in range(E_LOCAL):
                e = d * E_LOCAL + j
                xm = xc * (rc == e).astype(jnp.bfloat16)
                prod = jnp.dot(xm, w_ref[j], preferred_element_type=jnp.float32)
                total = prod if total is None else total + prod
            acc_ref[rows, :] = total.astype(jnp.bfloat16)

        for s in range(N_DEV - 1):
            if s == 0:
                send_chunk = d
                src = acc_ref.at[pl.ds(send_chunk * CHUNK, CHUNK), :]
            else:
                src = rs_comm.at[s - 1]
            rdma = pltpu.make_async_remote_copy(
                src_ref=src,
                dst_ref=rs_comm.at[s],
                send_sem=rs_send_sems.at[s],
                recv_sem=rs_recv_sems.at[s],
                device_id=(right,),
                device_id_type=pl.DeviceIdType.MESH,
            )
            rdma.start()
            rdma.wait()
            rchunk = lax.rem(d - s - 1 + N_DEV, N_DEV)
            rrows = pl.ds(rchunk * CHUNK, CHUNK)
            summed = rs_comm[s] + acc_ref[rrows, :]
            rs_comm[s] = summed
            if s == N_DEV - 2:
                out_ref[rrows, :] = summed.astype(jnp.float32)

        for s in range(N_DEV - 1):
            src = rs_comm.at[N_DEV - 2] if s == 0 else ag_comm.at[s - 1]
            rdma = pltpu.make_async_remote_copy(
                src_ref=src,
                dst_ref=ag_comm.at[s],
                send_sem=ag_send_sems.at[s],
                recv_sem=ag_recv_sems.at[s],
                device_id=(right,),
                device_id_type=pl.DeviceIdType.MESH,
            )
            rdma.start()
            rdma.wait()
            rchunk = lax.rem(d - s + N_DEV, N_DEV)
            out_ref[pl.ds(rchunk * CHUNK, CHUNK), :] = ag_comm[s].astype(
                jnp.float32
            )

    return pl.pallas_call(
        body,
        out_shape=jax.ShapeDtypeStruct((N_TOK, D_FF), jnp.float32),
        in_specs=[
            pl.BlockSpec(memory_space=pltpu.VMEM),
            pl.BlockSpec(memory_space=pltpu.VMEM),
            pl.BlockSpec(memory_space=pltpu.VMEM),
        ],
        out_specs=pl.BlockSpec(memory_space=pltpu.VMEM),
        scratch_shapes=[
            pltpu.VMEM((N_TOK, D_FF), jnp.bfloat16),
            pltpu.VMEM((N_DEV - 1, CHUNK, D_FF), jnp.bfloat16),
            pltpu.VMEM((N_DEV - 1, CHUNK, D_FF), jnp.bfloat16),
            pltpu.SemaphoreType.DMA((N_DEV - 1,)),
            pltpu.SemaphoreType.DMA((N_DEV - 1,)),
            pltpu.SemaphoreType.DMA((N_DEV - 1,)),
            pltpu.SemaphoreType.DMA((N_DEV - 1,)),
        ],
        compiler_params=pltpu.CompilerParams(collective_id=0),
    )(x_bf, route_idx, w_bf)


# baseline (device time: 474522 ns/iter reference)
import jax
import jax.numpy as jnp
from jax import lax
from jax.experimental import pallas as pl
from jax.experimental.pallas import tpu as pltpu

N_DEV = 8
N_TOK = 4096
D_MODEL = 1024
D_FF = 2048
E_LOCAL = 4
CHUNK = N_TOK // N_DEV


def kernel(x, router_W, route_idx, expert_W):
    x_bf = x.astype(jnp.bfloat16)
    w_bf = expert_W.astype(jnp.bfloat16)

    def body(
        x_ref,
        route_ref,
        w_ref,
        out_ref,
        own_ref,
        rs_comm,
        rs_send_sems,
        rs_recv_sems,
        ag_send_sems,
        ag_recv_sems,
        copy_sem,
        credit_sems,
    ):
        d = lax.axis_index("i")
        right = lax.rem(d + 1, N_DEV)
        left = lax.rem(d + N_DEV - 1, N_DEV)

        barrier_sem = pltpu.get_barrier_semaphore()
        for nbr in (left, right):
            pl.semaphore_signal(
                barrier_sem,
                inc=1,
                device_id=(nbr,),
                device_id_type=pl.DeviceIdType.MESH,
            )
        pl.semaphore_wait(barrier_sem, 2)

        def partial_prods(rows):
            xc = x_ref[rows, :]
            rc = route_ref[rows, :]
            for j in range(E_LOCAL):
                e = d * E_LOCAL + j
                xm = xc * (rc == e).astype(jnp.bfloat16)
                yield jnp.dot(
                    xm, w_ref[j], preferred_element_type=jnp.float32
                ).astype(jnp.bfloat16)

        for j, prod in enumerate(partial_prods(pl.ds(d * CHUNK, CHUNK))):
            own_ref[...] = prod if j == 0 else own_ref[...] + prod

        for s in range(N_DEV - 1):
            p = s % 2
            src = own_ref if s == 0 else rs_comm.at[(s - 1) % 2]
            if s >= 2:
                pl.semaphore_wait(credit_sems.at[p], 1)
            rdma = pltpu.make_async_remote_copy(
                src_ref=src,
                dst_ref=rs_comm.at[p],
                send_sem=rs_send_sems.at[p],
                recv_sem=rs_recv_sems.at[p],
                device_id=(right,),
                device_id_type=pl.DeviceIdType.MESH,
            )
            rdma.start()
            rdma.wait()
            if 1 <= s <= 5:
                pl.semaphore_signal(
                    credit_sems.at[(s - 1) % 2],
                    inc=1,
                    device_id=(left,),
                    device_id_type=pl.DeviceIdType.MESH,
                )
            rchunk = lax.rem(d - s - 1 + N_DEV, N_DEV)
            for prod in partial_prods(pl.ds(rchunk * CHUNK, CHUNK)):
                rs_comm[p] = rs_comm[p] + prod

        own_chunk = lax.rem(d + 1, N_DEV)
        cp = pltpu.make_async_copy(
            rs_comm.at[0],
            out_ref.at[pl.ds(own_chunk * CHUNK, CHUNK)],
            copy_sem,
        )
        cp.start()
        cp.wait()

        for s in range(N_DEV - 1):
            c = lax.rem(d + 1 - s + N_DEV, N_DEV)
            rows = pl.ds(c * CHUNK, CHUNK)
            rdma = pltpu.make_async_remote_copy(
                src_ref=out_ref.at[rows],
                dst_ref=out_ref.at[rows],
                send_sem=ag_send_sems.at[s],
                recv_sem=ag_recv_sems.at[s],
                device_id=(right,),
                device_id_type=pl.DeviceIdType.MESH,
            )
            rdma.start()
            rdma.wait()

    out_bf = pl.pallas_call(
        body,
        out_shape=jax.ShapeDtypeStruct((N_TOK, D_FF), jnp.bfloat16),
        in_specs=[
            pl.BlockSpec(memory_space=pltpu.VMEM),
            pl.BlockSpec(memory_space=pltpu.VMEM),
            pl.BlockSpec(memory_space=pltpu.VMEM),
        ],
        out_specs=pl.BlockSpec(memory_space=pl.ANY),
        scratch_shapes=[
            pltpu.VMEM((CHUNK, D_FF), jnp.bfloat16),
            pltpu.VMEM((2, CHUNK, D_FF), jnp.bfloat16),
            pltpu.SemaphoreType.DMA((2,)),
            pltpu.SemaphoreType.DMA((2,)),
            pltpu.SemaphoreType.DMA((N_DEV - 1,)),
            pltpu.SemaphoreType.DMA((N_DEV - 1,)),
            pltpu.SemaphoreType.DMA,
            pltpu.SemaphoreType.REGULAR((2,)),
        ],
        compiler_params=pltpu.CompilerParams(collective_id=0),
    )(x_bf, route_idx, w_bf)
    return out_bf.astype(jnp.float32)


# device time: 387865 ns/iter; 1.2234x vs baseline; 1.2234x over previous
import jax
import jax.numpy as jnp
from jax import lax
from jax.experimental import pallas as pl
from jax.experimental.pallas import tpu as pltpu

N_DEV = 8
N_TOK = 4096
D_MODEL = 1024
D_FF = 2048
E_LOCAL = 4
CHUNK = N_TOK // N_DEV


def kernel(x, router_W, route_idx, expert_W):
    x_bf = x.astype(jnp.bfloat16)
    w_bf = expert_W.astype(jnp.bfloat16)

    def body(
        x_ref,
        route_ref,
        w_ref,
        out_ref,
        slots_ref,
        sbuf,
        acc,
        stag,
        rs_send_sems,
        rs_recv_sems,
        ag_send_sems,
        ag_recv_sems,
        copy_sems,
    ):
        d = lax.axis_index("i")

        barrier_sem = pltpu.get_barrier_semaphore()
        for k in range(1, N_DEV):
            pl.semaphore_signal(
                barrier_sem,
                inc=1,
                device_id=(lax.rem(d + k, N_DEV),),
                device_id_type=pl.DeviceIdType.MESH,
            )
        pl.semaphore_wait(barrier_sem, N_DEV - 1)

        def accum_partial(dst_write, rows):
            xc = x_ref[rows, :]
            rc = route_ref[rows, :]
            for j in range(E_LOCAL):
                e = d * E_LOCAL + j
                xm = xc * (rc == e).astype(jnp.bfloat16)
                prod = jnp.dot(
                    xm, w_ref[j], preferred_element_type=jnp.float32
                ).astype(jnp.bfloat16)
                dst_write(j, prod)

        rs_sends = []
        for k in range(1, N_DEV):
            o = lax.rem(d + k, N_DEV)
            p = k % 2
            if k >= 3:
                rs_sends[k - 3].wait_send()

            def wr(j, prod, _p=p):
                sbuf[_p] = prod if j == 0 else sbuf[_p] + prod

            accum_partial(wr, pl.ds(o * CHUNK, CHUNK))
            rdma = pltpu.make_async_remote_copy(
                src_ref=sbuf.at[p],
                dst_ref=slots_ref.at[N_DEV - 1 - k],
                send_sem=rs_send_sems.at[N_DEV - 1 - k],
                recv_sem=rs_recv_sems.at[N_DEV - 1 - k],
                device_id=(o,),
                device_id_type=pl.DeviceIdType.MESH,
            )
            rdma.start()
            rs_sends.append(rdma)

        def wr_acc(j, prod):
            acc[...] = prod if j == 0 else acc[...] + prod

        accum_partial(wr_acc, pl.ds(d * CHUNK, CHUNK))

        for q in range(N_DEV - 2, -1, -1):
            recv = pltpu.make_async_remote_copy(
                src_ref=sbuf.at[0],
                dst_ref=slots_ref.at[q],
                send_sem=rs_send_sems.at[q],
                recv_sem=rs_recv_sems.at[q],
                device_id=(d,),
                device_id_type=pl.DeviceIdType.MESH,
            )
            recv.wait_recv()
            cp = pltpu.make_async_copy(
                slots_ref.at[q], stag.at[q % 2], copy_sems.at[q % 2]
            )
            cp.start()
            cp.wait()
            acc[...] = acc[...] + stag[q % 2]

        rs_sends[-2].wait_send()
        rs_sends[-1].wait_send()

        my_rows = pl.ds(d * CHUNK, CHUNK)
        final_cp = pltpu.make_async_copy(
            acc, out_ref.at[my_rows], copy_sems.at[0]
        )
        final_cp.start()
        ag_sends = []
        for k in range(1, N_DEV):
            o = lax.rem(d + k, N_DEV)
            rdma = pltpu.make_async_remote_copy(
                src_ref=acc,
                dst_ref=out_ref.at[my_rows],
                send_sem=ag_send_sems.at[N_DEV - 1 - k],
                recv_sem=ag_recv_sems.at[N_DEV - 1 - k],
                device_id=(o,),
                device_id_type=pl.DeviceIdType.MESH,
            )
            rdma.start()
            ag_sends.append(rdma)

        for q in range(N_DEV - 2, -1, -1):
            src_dev = lax.rem(d + q + 1, N_DEV)
            recv = pltpu.make_async_remote_copy(
                src_ref=sbuf.at[0],
                dst_ref=out_ref.at[pl.ds(src_dev * CHUNK, CHUNK)],
                send_sem=ag_send_sems.at[q],
                recv_sem=ag_recv_sems.at[q],
                device_id=(d,),
                device_id_type=pl.DeviceIdType.MESH,
            )
            recv.wait_recv()

        for rdma in ag_sends:
            rdma.wait_send()
        final_cp.wait()

    out_bf, _ = pl.pallas_call(
        body,
        out_shape=(
            jax.ShapeDtypeStruct((N_TOK, D_FF), jnp.bfloat16),
            jax.ShapeDtypeStruct((N_DEV - 1, CHUNK, D_FF), jnp.bfloat16),
        ),
        in_specs=[
            pl.BlockSpec(memory_space=pltpu.VMEM),
            pl.BlockSpec(memory_space=pltpu.VMEM),
            pl.BlockSpec(memory_space=pltpu.VMEM),
        ],
        out_specs=(
            pl.BlockSpec(memory_space=pl.ANY),
            pl.BlockSpec(memory_space=pl.ANY),
        ),
        scratch_shapes=[
            pltpu.VMEM((2, CHUNK, D_FF), jnp.bfloat16),
            pltpu.VMEM((CHUNK, D_FF), jnp.bfloat16),
            pltpu.VMEM((2, CHUNK, D_FF), jnp.bfloat16),
            pltpu.SemaphoreType.DMA((N_DEV - 1,)),
            pltpu.SemaphoreType.DMA((N_DEV - 1,)),
            pltpu.SemaphoreType.DMA((N_DEV - 1,)),
            pltpu.SemaphoreType.DMA((N_DEV - 1,)),
            pltpu.SemaphoreType.DMA((2,)),
        ],
        compiler_params=pltpu.CompilerParams(collective_id=0),
    )(x_bf, route_idx, w_bf)
    return out_bf.astype(jnp.float32)
